# Initial kernel scaffold; baseline (speedup 1.0000x reference)
import jax
import jax.numpy as jnp
from jax.experimental import pallas as pl


def kernel(input_ids):
    raise NotImplementedError("write your pallas kernel here")


# speedup vs baseline: 1.2248x; 1.2248x over previous
import jax
import jax.numpy as jnp
from jax.experimental import pallas as pl

VOCAB = 1000
ROWS_PER_BLOCK = 512


def _onehot_block(ids_ref, out_ref):
    ids = ids_ref[0, 0, :]
    iota = jax.lax.broadcasted_iota(jnp.int32, (ROWS_PER_BLOCK, VOCAB), 1)
    out_ref[...] = (iota == ids[:, None]).astype(jnp.float32)


def kernel(input_ids) -> jnp.ndarray:
    B, L = input_ids.shape
    n = B * L
    nb = n // ROWS_PER_BLOCK
    ids_flat = input_ids.reshape(nb, 1, ROWS_PER_BLOCK).astype(jnp.int32)
    out = pl.pallas_call(
        _onehot_block,
        grid=(nb,),
        in_specs=[pl.BlockSpec((1, 1, ROWS_PER_BLOCK), lambda i: (i, 0, 0))],
        out_specs=pl.BlockSpec((ROWS_PER_BLOCK, VOCAB), lambda i: (i, 0)),
        out_shape=jax.ShapeDtypeStruct((n, VOCAB), jnp.float32),
    )(ids_flat)
    return out.reshape(B, L, VOCAB)


# speedup vs baseline: 1.2878x; 1.0514x over previous
import jax
import jax.numpy as jnp
from jax.experimental import pallas as pl

VOCAB = 1000
ROWS_PER_BLOCK = 2048


def _onehot_block(ids_ref, out_ref):
    ids = ids_ref[0, 0, :]
    iota = jax.lax.broadcasted_iota(jnp.int32, (ROWS_PER_BLOCK, VOCAB), 1)
    out_ref[...] = (iota == ids[:, None]).astype(jnp.float32)


def kernel(input_ids) -> jnp.ndarray:
    B, L = input_ids.shape
    n = B * L
    nb = n // ROWS_PER_BLOCK
    ids_flat = input_ids.reshape(nb, 1, ROWS_PER_BLOCK).astype(jnp.int32)
    out = pl.pallas_call(
        _onehot_block,
        grid=(nb,),
        in_specs=[pl.BlockSpec((1, 1, ROWS_PER_BLOCK), lambda i: (i, 0, 0))],
        out_specs=pl.BlockSpec((ROWS_PER_BLOCK, VOCAB), lambda i: (i, 0)),
        out_shape=jax.ShapeDtypeStruct((n, VOCAB), jnp.float32),
    )(ids_flat)
    return out.reshape(B, L, VOCAB)
